# baseline (speedup 1.0000x reference)
import functools

import jax
import jax.numpy as jnp
from jax import lax
from jax.experimental import pallas as pl
from jax.experimental.pallas import tpu as pltpu
from jax.experimental.pallas import tpu_sc as plsc

EMBED = 64
NSPLIT = 16


def kernel(next_id, next_category, next_subcategory, id_table, category_table,
           subcategory_table):
    B = next_id.shape[0]
    next_id = next_id.astype(jnp.int32)
    next_category = next_category.astype(jnp.int32)
    next_subcategory = next_subcategory.astype(jnp.int32)

    info = plsc.get_sparse_core_info()
    nw = info.num_cores * info.num_subcores
    b_per_w = B // nw
    piece = b_per_w // NSPLIT

    mesh = plsc.VectorSubcoreMesh(core_axis_name="c", subcore_axis_name="s")

    @functools.partial(
        pl.kernel,
        mesh=mesh,
        out_type=jax.ShapeDtypeStruct((B, 3 * EMBED), jnp.float32),
        compiler_params=pltpu.CompilerParams(use_tc_tiling_on_sc=False),
        scratch_types=[
            pltpu.VMEM((b_per_w,), jnp.int32),
            pltpu.VMEM((b_per_w,), jnp.int32),
            pltpu.VMEM((b_per_w,), jnp.int32),
            pltpu.VMEM((b_per_w, EMBED), jnp.float32),
            pltpu.VMEM((b_per_w, EMBED), jnp.float32),
            pltpu.VMEM((b_per_w, EMBED), jnp.float32),
            [pltpu.SemaphoreType.DMA for _ in range(NSPLIT)],
            [pltpu.SemaphoreType.DMA for _ in range(2)],
            [pltpu.SemaphoreType.DMA for _ in range(3)],
            pltpu.SemaphoreType.DMA,
        ],
    )
    def gather_concat(id_idx_hbm, cat_idx_hbm, sub_idx_hbm, id_tab, cat_tab,
                      sub_tab, out_hbm, idx0, idx1, idx2, rows0, rows1, rows2,
                      gsem0, gsem12, wsem, isem):
        wid = lax.axis_index("s") * info.num_cores + lax.axis_index("c")
        base = wid * b_per_w
        i0 = pltpu.async_copy(id_idx_hbm.at[pl.ds(base, b_per_w)], idx0, isem)
        i1 = pltpu.async_copy(cat_idx_hbm.at[pl.ds(base, b_per_w)], idx1, isem)
        i2 = pltpu.async_copy(sub_idx_hbm.at[pl.ds(base, b_per_w)], idx2, isem)
        i0.wait(); i1.wait(); i2.wait()
        id_gathers = [
            pltpu.async_copy(
                id_tab.at[idx0.at[pl.ds(k * piece, piece)]],
                rows0.at[pl.ds(k * piece, piece)], gsem0[k])
            for k in range(NSPLIT)
        ]
        g1 = pltpu.async_copy(cat_tab.at[idx1], rows1, gsem12[0])
        g2 = pltpu.async_copy(sub_tab.at[idx2], rows2, gsem12[1])
        g1.wait()
        w1 = pltpu.async_copy(
            rows1, out_hbm.at[pl.ds(base, b_per_w), pl.ds(EMBED, EMBED)],
            wsem[1])
        g2.wait()
        w2 = pltpu.async_copy(
            rows2, out_hbm.at[pl.ds(base, b_per_w), pl.ds(2 * EMBED, EMBED)],
            wsem[2])
        for g in id_gathers:
            g.wait()
        w0 = pltpu.async_copy(
            rows0, out_hbm.at[pl.ds(base, b_per_w), pl.ds(0, EMBED)], wsem[0])
        w1.wait()
        w2.wait()
        w0.wait()

    return gather_concat(next_id, next_category, next_subcategory, id_table,
                         category_table, subcategory_table)


# speedup vs baseline: 1.0058x; 1.0058x over previous
import functools

import jax
import jax.numpy as jnp
from jax import lax
from jax.experimental import pallas as pl
from jax.experimental.pallas import tpu as pltpu
from jax.experimental.pallas import tpu_sc as plsc

EMBED = 64
NSPLIT = 8


def kernel(next_id, next_category, next_subcategory, id_table, category_table,
           subcategory_table):
    B = next_id.shape[0]
    next_id = next_id.astype(jnp.int32)
    next_category = next_category.astype(jnp.int32)
    next_subcategory = next_subcategory.astype(jnp.int32)

    info = plsc.get_sparse_core_info()
    nw = info.num_cores * info.num_subcores
    b_per_w = B // nw
    piece = b_per_w // NSPLIT

    mesh = plsc.VectorSubcoreMesh(core_axis_name="c", subcore_axis_name="s")

    @functools.partial(
        pl.kernel,
        mesh=mesh,
        out_type=jax.ShapeDtypeStruct((B, 3 * EMBED), jnp.float32),
        compiler_params=pltpu.CompilerParams(use_tc_tiling_on_sc=False),
        scratch_types=[
            pltpu.VMEM((b_per_w,), jnp.int32),
            pltpu.VMEM((b_per_w,), jnp.int32),
            pltpu.VMEM((b_per_w,), jnp.int32),
            pltpu.VMEM((b_per_w, EMBED), jnp.float32),
            pltpu.VMEM((b_per_w, EMBED), jnp.float32),
            pltpu.VMEM((b_per_w, EMBED), jnp.float32),
            [pltpu.SemaphoreType.DMA for _ in range(NSPLIT)],
            [pltpu.SemaphoreType.DMA for _ in range(2)],
            [pltpu.SemaphoreType.DMA for _ in range(3)],
            pltpu.SemaphoreType.DMA,
        ],
    )
    def gather_concat(id_idx_hbm, cat_idx_hbm, sub_idx_hbm, id_tab, cat_tab,
                      sub_tab, out_hbm, idx0, idx1, idx2, rows0, rows1, rows2,
                      gsem0, gsem12, wsem, isem):
        wid = lax.axis_index("s") * info.num_cores + lax.axis_index("c")
        base = wid * b_per_w
        i0 = pltpu.async_copy(id_idx_hbm.at[pl.ds(base, b_per_w)], idx0, isem)
        i1 = pltpu.async_copy(cat_idx_hbm.at[pl.ds(base, b_per_w)], idx1, isem)
        i2 = pltpu.async_copy(sub_idx_hbm.at[pl.ds(base, b_per_w)], idx2, isem)
        i0.wait(); i1.wait(); i2.wait()
        id_gathers = [
            pltpu.async_copy(
                id_tab.at[idx0.at[pl.ds(k * piece, piece)]],
                rows0.at[pl.ds(k * piece, piece)], gsem0[k])
            for k in range(NSPLIT)
        ]
        g1 = pltpu.async_copy(cat_tab.at[idx1], rows1, gsem12[0])
        g2 = pltpu.async_copy(sub_tab.at[idx2], rows2, gsem12[1])
        g1.wait()
        w1 = pltpu.async_copy(
            rows1, out_hbm.at[pl.ds(base, b_per_w), pl.ds(EMBED, EMBED)],
            wsem[1])
        g2.wait()
        w2 = pltpu.async_copy(
            rows2, out_hbm.at[pl.ds(base, b_per_w), pl.ds(2 * EMBED, EMBED)],
            wsem[2])
        for g in id_gathers:
            g.wait()
        w0 = pltpu.async_copy(
            rows0, out_hbm.at[pl.ds(base, b_per_w), pl.ds(0, EMBED)], wsem[0])
        w1.wait()
        w2.wait()
        w0.wait()

    return gather_concat(next_id, next_category, next_subcategory, id_table,
                         category_table, subcategory_table)


# speedup vs baseline: 1.3380x; 1.3303x over previous
import functools

import jax
import jax.numpy as jnp
from jax import lax
from jax.experimental import pallas as pl
from jax.experimental.pallas import tpu as pltpu
from jax.experimental.pallas import tpu_sc as plsc

EMBED = 64
NSPLIT = 4


def kernel(next_id, next_category, next_subcategory, id_table, category_table,
           subcategory_table):
    B = next_id.shape[0]
    next_id = next_id.astype(jnp.int32)
    next_category = next_category.astype(jnp.int32)
    next_subcategory = next_subcategory.astype(jnp.int32)
    cat_rows = category_table.shape[0]
    sub_rows = subcategory_table.shape[0]

    info = plsc.get_sparse_core_info()
    nw = info.num_cores * info.num_subcores
    b_per_w = B // nw
    piece = b_per_w // NSPLIT

    mesh = plsc.VectorSubcoreMesh(core_axis_name="c", subcore_axis_name="s")

    @functools.partial(
        pl.kernel,
        mesh=mesh,
        out_type=jax.ShapeDtypeStruct((B, 3 * EMBED), jnp.float32),
        compiler_params=pltpu.CompilerParams(use_tc_tiling_on_sc=False),
        scratch_types=[
            pltpu.VMEM((b_per_w,), jnp.int32),
            pltpu.VMEM((b_per_w,), jnp.int32),
            pltpu.VMEM((b_per_w,), jnp.int32),
            pltpu.VMEM((b_per_w, EMBED), jnp.float32),
            pltpu.VMEM((b_per_w, EMBED), jnp.float32),
            pltpu.VMEM((b_per_w, EMBED), jnp.float32),
            pltpu.VMEM_SHARED((cat_rows, EMBED), jnp.float32),
            pltpu.VMEM_SHARED((sub_rows, EMBED), jnp.float32),
            [pltpu.SemaphoreType.DMA for _ in range(NSPLIT)],
            [pltpu.SemaphoreType.DMA for _ in range(2)],
            [pltpu.SemaphoreType.DMA for _ in range(3)],
            pltpu.SemaphoreType.DMA,
        ],
    )
    def gather_concat(id_idx_hbm, cat_idx_hbm, sub_idx_hbm, id_tab, cat_tab,
                      sub_tab, out_hbm, idx0, idx1, idx2, rows0, rows1, rows2,
                      cat_sh, sub_sh, gsem0, gsem12, wsem, isem):
        sid = lax.axis_index("s")
        wid = sid * info.num_cores + lax.axis_index("c")
        base = wid * b_per_w
        i0 = pltpu.async_copy(id_idx_hbm.at[pl.ds(base, b_per_w)], idx0, isem)
        i1 = pltpu.async_copy(cat_idx_hbm.at[pl.ds(base, b_per_w)], idx1, isem)
        i2 = pltpu.async_copy(sub_idx_hbm.at[pl.ds(base, b_per_w)], idx2, isem)

        @pl.when(sid == 0)
        def _stage():
            pltpu.sync_copy(cat_tab, cat_sh)
            pltpu.sync_copy(sub_tab, sub_sh)

        i0.wait()
        id_gathers = [
            pltpu.async_copy(
                id_tab.at[idx0.at[pl.ds(k * piece, piece)]],
                rows0.at[pl.ds(k * piece, piece)], gsem0[k])
            for k in range(NSPLIT)
        ]
        plsc.subcore_barrier()
        i1.wait(); i2.wait()
        g1 = pltpu.async_copy(cat_sh.at[idx1], rows1, gsem12[0])
        g2 = pltpu.async_copy(sub_sh.at[idx2], rows2, gsem12[1])
        g1.wait()
        w1 = pltpu.async_copy(
            rows1, out_hbm.at[pl.ds(base, b_per_w), pl.ds(EMBED, EMBED)],
            wsem[1])
        g2.wait()
        w2 = pltpu.async_copy(
            rows2, out_hbm.at[pl.ds(base, b_per_w), pl.ds(2 * EMBED, EMBED)],
            wsem[2])
        for g in id_gathers:
            g.wait()
        w0 = pltpu.async_copy(
            rows0, out_hbm.at[pl.ds(base, b_per_w), pl.ds(0, EMBED)], wsem[0])
        w1.wait()
        w2.wait()
        w0.wait()

    return gather_concat(next_id, next_category, next_subcategory, id_table,
                         category_table, subcategory_table)
